# Initial kernel scaffold; baseline (speedup 1.0000x reference)
import jax
import jax.numpy as jnp
from jax.experimental import pallas as pl


def kernel(xs0, W0, b0, xs1, W1, b1, xs2, W2, b2, xs3, W3, b3, xs4, W4, b4, xs5, W5, b5, xs6, W6, b6, xs7, W7, b7, gates):
    raise NotImplementedError("write your pallas kernel here")


# speedup vs baseline: 1.2729x; 1.2729x over previous
import functools

import jax
import jax.numpy as jnp
import numpy as np
from jax.experimental import pallas as pl
from jax.experimental.pallas import tpu as pltpu

B, C, L, D, E, P = 32, 16, 16, 512, 8, 720
_EPS = float(np.finfo(np.float64).eps)


def _lph_kernel(x_ref, w_ref, b_ref, g_ref, out_ref, acc_ref):
    e = pl.program_id(0)
    x = x_ref[0]
    w = w_ref[0]
    bias = b_ref[0]
    g = g_ref[0]
    y = jax.lax.dot_general(
        x, w, (((1,), (1,)), ((), ())), preferred_element_type=jnp.float32
    )
    term = jnp.exp(y + bias) * g

    @pl.when(e == 0)
    def _init():
        acc_ref[...] = term

    @pl.when(e != 0)
    def _acc():
        acc_ref[...] = acc_ref[...] + term

    @pl.when(e == E - 1)
    def _finish():
        acc = acc_ref[...]
        res = jnp.log(jnp.where(acc == 0.0, _EPS, acc))
        out_ref[...] = jnp.transpose(res.reshape(B, C, P), (0, 2, 1))


@jax.jit
def kernel(xs0, W0, b0, xs1, W1, b1, xs2, W2, b2, xs3, W3, b3,
           xs4, W4, b4, xs5, W5, b5, xs6, W6, b6, xs7, W7, b7, gates):
    xs = [xs0, xs1, xs2, xs3, xs4, xs5, xs6, xs7]
    Ws = [W0, W1, W2, W3, W4, W5, W6, W7]
    bs = [b0, b1, b2, b3, b4, b5, b6, b7]

    X = jnp.stack([x[:, :, L - 1, :].reshape(B * C, D) for x in xs])
    W = jnp.stack(Ws)
    bias = jnp.stack(bs).reshape(E, 1, P)
    g_rows = jnp.repeat(gates, C, axis=0).T.reshape(E, B * C, 1)

    out = pl.pallas_call(
        _lph_kernel,
        grid=(E,),
        in_specs=[
            pl.BlockSpec((1, B * C, D), lambda e: (e, 0, 0)),
            pl.BlockSpec((1, P, D), lambda e: (e, 0, 0)),
            pl.BlockSpec((1, 1, P), lambda e: (e, 0, 0)),
            pl.BlockSpec((1, B * C, 1), lambda e: (e, 0, 0)),
        ],
        out_specs=pl.BlockSpec((B, P, C), lambda e: (0, 0, 0)),
        out_shape=jax.ShapeDtypeStruct((B, P, C), jnp.float32),
        scratch_shapes=[pltpu.VMEM((B * C, P), jnp.float32)],
    )(X, W, bias, g_rows)
    return out


# speedup vs baseline: 3.5401x; 2.7810x over previous
import jax
import jax.numpy as jnp
import numpy as np
from jax.experimental import pallas as pl
from jax.experimental.pallas import tpu as pltpu

B, C, L, D, E, P = 32, 16, 16, 512, 8, 720
_EPS = float(np.finfo(np.float64).eps)


def _lph_kernel(*refs):
    xs_refs = refs[0:E]
    w_refs = refs[E:2 * E]
    b_ref = refs[2 * E]
    g_ref = refs[2 * E + 1]
    out_ref = refs[2 * E + 2]
    x_scr = refs[2 * E + 3]
    w_scr = refs[2 * E + 4]
    sem = refs[2 * E + 5]

    def x_copy(e):
        return pltpu.make_async_copy(
            xs_refs[e].at[:, :, pl.ds(L - 1, 1), :], x_scr.at[e], sem.at[2 * e]
        )

    def w_copy(e):
        return pltpu.make_async_copy(w_refs[e], w_scr.at[e], sem.at[2 * e + 1])

    for e in range(E):
        x_copy(e).start()
        w_copy(e).start()

    acc = None
    for e in range(E):
        x_copy(e).wait()
        w_copy(e).wait()
        x = x_scr[e].reshape(B * C, D)
        w = w_scr[e]
        y = jax.lax.dot_general(
            x, w, (((1,), (1,)), ((), ())), preferred_element_type=jnp.float32
        )
        term = jnp.exp(y + b_ref[e]) * g_ref[e]
        acc = term if acc is None else acc + term

    res = jnp.log(jnp.where(acc == 0.0, _EPS, acc))
    out_ref[...] = jnp.transpose(res.reshape(B, C, P), (0, 2, 1))


@jax.jit
def kernel(xs0, W0, b0, xs1, W1, b1, xs2, W2, b2, xs3, W3, b3,
           xs4, W4, b4, xs5, W5, b5, xs6, W6, b6, xs7, W7, b7, gates):
    xs = [xs0, xs1, xs2, xs3, xs4, xs5, xs6, xs7]
    Ws = [W0, W1, W2, W3, W4, W5, W6, W7]
    bias = jnp.stack([b0, b1, b2, b3, b4, b5, b6, b7]).reshape(E, 1, P)
    g_rows = jnp.repeat(gates, C, axis=0).T.reshape(E, B * C, 1)

    any_spec = pl.BlockSpec(memory_space=pltpu.MemorySpace.HBM)

    out = pl.pallas_call(
        _lph_kernel,
        in_specs=[any_spec] * (2 * E) + [
            pl.BlockSpec((E, 1, P), lambda: (0, 0, 0)),
            pl.BlockSpec((E, B * C, 1), lambda: (0, 0, 0)),
        ],
        out_specs=pl.BlockSpec((B, P, C), lambda: (0, 0, 0)),
        out_shape=jax.ShapeDtypeStruct((B, P, C), jnp.float32),
        scratch_shapes=[
            pltpu.VMEM((E, B, C, 1, D), jnp.float32),
            pltpu.VMEM((E, P, D), jnp.float32),
            pltpu.SemaphoreType.DMA((2 * E,)),
        ],
    )(*xs, *Ws, bias, g_rows)
    return out
